# baseline (speedup 1.0000x reference)
import functools

import jax
import jax.numpy as jnp
from jax import lax
from jax.experimental import pallas as pl
from jax.experimental.pallas import tpu as pltpu
from jax.experimental.pallas import tpu_sc as plsc

N_HASHES = 2
N_HEADS = 8
PADDED = 4096
BLOCK = 64
DPH = 32
CDIM = 3
D = DPH + CDIM


def _hash_body(q_ref, k_ref, al_ref, cs_ref, qh_ref, kh_ref):
    q = q_ref[0]
    k = k_ref[0]
    al = al_ref[0]
    cs = cs_ref[0].astype(jnp.float32)
    qh = jax.lax.dot_general(q, al, (((1,), (0,)), ((), ())),
                             preferred_element_type=jnp.float32, precision=jax.lax.Precision.HIGHEST)
    kh = jax.lax.dot_general(k, al, (((1,), (0,)), ((), ())),
                             preferred_element_type=jnp.float32, precision=jax.lax.Precision.HIGHEST)
    mx = jnp.maximum(jnp.max(qh, axis=0), jnp.max(kh, axis=0))
    mn = jnp.minimum(jnp.min(qh, axis=0), jnp.min(kh, axis=0))
    shift = cs * (mx - mn)[None, :]
    qh_ref[0] = qh + shift
    kh_ref[0] = kh + shift


def _hash_stage(query, key, alpha, cs_t):
    out = pl.pallas_call(
        _hash_body,
        grid=(N_HEADS,),
        in_specs=[
            pl.BlockSpec((1, PADDED, D), lambda h: (h, 0, 0)),
            pl.BlockSpec((1, PADDED, D), lambda h: (h, 0, 0)),
            pl.BlockSpec((1, D, N_HASHES), lambda h: (h, 0, 0)),
            pl.BlockSpec((1, PADDED, N_HASHES), lambda h: (h, 0, 0)),
        ],
        out_specs=[
            pl.BlockSpec((1, PADDED, N_HASHES), lambda h: (h, 0, 0)),
            pl.BlockSpec((1, PADDED, N_HASHES), lambda h: (h, 0, 0)),
        ],
        out_shape=[
            jax.ShapeDtypeStruct((N_HEADS, PADDED, N_HASHES), jnp.float32),
            jax.ShapeDtypeStruct((N_HEADS, PADDED, N_HASHES), jnp.float32),
        ],
    )(query, key, alpha, cs_t)
    return out


NW = 32
ROWS_PER_W = (N_HASHES * N_HEADS * PADDED) // NW
GCH = 512
DPAD = 48


def _gather_body(q2d, k2d, v2d, qg, kg, sq, sk, sv,
                 idxq_v, idxk_v, rows_q, rows_k, rows_v, sem):
    wid = lax.axis_index("s") * 2 + lax.axis_index("c")
    base = wid * ROWS_PER_W
    for c in range(ROWS_PER_W // GCH):
        off = base + c * GCH
        pltpu.sync_copy(qg.at[pl.ds(off, GCH)], idxq_v)
        pltpu.sync_copy(kg.at[pl.ds(off, GCH)], idxk_v)
        pltpu.async_copy(q2d.at[idxq_v], rows_q, sem).wait()
        pltpu.sync_copy(rows_q, sq.at[pl.ds(off, GCH)])
        pltpu.async_copy(k2d.at[idxk_v], rows_k, sem).wait()
        pltpu.sync_copy(rows_k, sk.at[pl.ds(off, GCH)])
        pltpu.async_copy(v2d.at[idxk_v], rows_v, sem).wait()
        pltpu.sync_copy(rows_v, sv.at[pl.ds(off, GCH)])


def _gather_stage(query, key, value, q_pos, k_pos):
    pad = ((0, 0), (0, 0), (0, DPAD - D))
    q2d = jnp.pad(query, pad).reshape(N_HEADS * PADDED, DPAD)
    k2d = jnp.pad(key, pad).reshape(N_HEADS * PADDED, DPAD)
    v2d = value.reshape(N_HEADS * PADDED, DPH)
    head_off = (jnp.arange(N_HEADS, dtype=jnp.int32) * PADDED)[None, :, None]
    qg = (q_pos + head_off).reshape(-1)
    kg = (k_pos + head_off).reshape(-1)
    run = pl.kernel(
        _gather_body,
        out_type=[
            jax.ShapeDtypeStruct((N_HASHES * N_HEADS * PADDED, DPAD), jnp.float32),
            jax.ShapeDtypeStruct((N_HASHES * N_HEADS * PADDED, DPAD), jnp.float32),
            jax.ShapeDtypeStruct((N_HASHES * N_HEADS * PADDED, DPH), jnp.float32),
        ],
        mesh=plsc.VectorSubcoreMesh(core_axis_name="c", subcore_axis_name="s",
                                    num_cores=2, num_subcores=16),
        compiler_params=pltpu.CompilerParams(use_tc_tiling_on_sc=False),
        scratch_types=[
            pltpu.VMEM((GCH,), jnp.int32),
            pltpu.VMEM((GCH,), jnp.int32),
            pltpu.VMEM((GCH, DPAD), jnp.float32),
            pltpu.VMEM((GCH, DPAD), jnp.float32),
            pltpu.VMEM((GCH, DPH), jnp.float32),
            pltpu.SemaphoreType.DMA,
        ],
    )
    sq2d, sk2d, sv2d = run(q2d, k2d, v2d, qg, kg)
    shp = (N_HASHES, N_HEADS, PADDED)
    return (sq2d.reshape(shp + (DPAD,)), sk2d.reshape(shp + (DPAD,)),
            sv2d.reshape(shp + (DPH,)))


BLOCKS_PER_STEP = 8
ROWS_PER_STEP = BLOCKS_PER_STEP * BLOCK


def _attn_body(q_ref, k_ref, v_ref, o_ref):
    q = q_ref[0, 0].reshape(BLOCKS_PER_STEP, BLOCK, DPAD)
    k = k_ref[0, 0].reshape(BLOCKS_PER_STEP, BLOCK, DPAD)
    v = v_ref[0, 0].reshape(BLOCKS_PER_STEP, BLOCK, DPH)
    qn = jnp.sum(q * q, axis=-1)
    kn = jnp.sum(k * k, axis=-1)
    qk = jax.lax.dot_general(q, k, (((2,), (2,)), ((0,), (0,))),
                             preferred_element_type=jnp.float32, precision=jax.lax.Precision.HIGHEST)
    d2 = qn[:, :, None] + kn[:, None, :] - 2.0 * qk
    w = jnp.exp(-0.5 * d2)
    o = jax.lax.dot_general(w, v, (((2,), (1,)), ((0,), (0,))),
                            preferred_element_type=jnp.float32, precision=jax.lax.Precision.HIGHEST)
    o_ref[0, 0] = o


def _attn_stage(sq, sk, sv):
    n_steps = PADDED // ROWS_PER_STEP
    out = pl.pallas_call(
        _attn_body,
        grid=(N_HASHES, N_HEADS, n_steps),
        in_specs=[
            pl.BlockSpec((1, 1, ROWS_PER_STEP, DPAD), lambda a, h, b: (a, h, b, 0)),
            pl.BlockSpec((1, 1, ROWS_PER_STEP, DPAD), lambda a, h, b: (a, h, b, 0)),
            pl.BlockSpec((1, 1, ROWS_PER_STEP, DPH), lambda a, h, b: (a, h, b, 0)),
        ],
        out_specs=pl.BlockSpec((1, 1, BLOCKS_PER_STEP, BLOCK, DPH),
                               lambda a, h, b: (a, h, b, 0, 0)),
        out_shape=jax.ShapeDtypeStruct(
            (N_HASHES, N_HEADS, PADDED // BLOCK, BLOCK, DPH), jnp.float32),
    )(sq, sk, sv)
    return out


def kernel(query, key, value, combined_shifts, alpha):
    q_hashed = jnp.einsum('hpd,hdn->hpn', query, alpha).transpose(2, 0, 1)
    k_hashed = jnp.einsum('hpd,hdn->hpn', key, alpha).transpose(2, 0, 1)
    max_hash_shift = jnp.maximum(q_hashed.max(-1, keepdims=True),
                                 k_hashed.max(-1, keepdims=True))
    min_hash_shift = jnp.minimum(q_hashed.min(-1, keepdims=True),
                                 k_hashed.min(-1, keepdims=True))
    hash_shift = max_hash_shift - min_hash_shift
    cs = combined_shifts.astype(jnp.float32) * hash_shift
    q_keys = q_hashed + cs
    k_keys = k_hashed + cs
    q_pos = jnp.argsort(q_keys, axis=-1).astype(jnp.int32)
    k_pos = jnp.argsort(k_keys, axis=-1).astype(jnp.int32)
    sq, sk, sv = _gather_stage(query, key, value, q_pos, k_pos)
    return _attn_stage(sq, sk, sv)


# speedup vs baseline: 1.4096x; 1.4096x over previous
import functools

import jax
import jax.numpy as jnp
from jax import lax
from jax.experimental import pallas as pl
from jax.experimental.pallas import tpu as pltpu
from jax.experimental.pallas import tpu_sc as plsc

N_HASHES = 2
N_HEADS = 8
PADDED = 4096
BLOCK = 64
DPH = 32
CDIM = 3
D = DPH + CDIM


def _hash_body(q_ref, k_ref, al_ref, cs_ref, qh_ref, kh_ref):
    q = q_ref[0]
    k = k_ref[0]
    al = al_ref[0]
    cs = cs_ref[0].astype(jnp.float32)
    qh = jax.lax.dot_general(q, al, (((1,), (0,)), ((), ())),
                             preferred_element_type=jnp.float32, precision=jax.lax.Precision.HIGHEST)
    kh = jax.lax.dot_general(k, al, (((1,), (0,)), ((), ())),
                             preferred_element_type=jnp.float32, precision=jax.lax.Precision.HIGHEST)
    mx = jnp.maximum(jnp.max(qh, axis=0), jnp.max(kh, axis=0))
    mn = jnp.minimum(jnp.min(qh, axis=0), jnp.min(kh, axis=0))
    shift = cs * (mx - mn)[None, :]
    qh_ref[0] = qh + shift
    kh_ref[0] = kh + shift


def _hash_stage(query, key, alpha, cs_t):
    out = pl.pallas_call(
        _hash_body,
        grid=(N_HEADS,),
        in_specs=[
            pl.BlockSpec((1, PADDED, D), lambda h: (h, 0, 0)),
            pl.BlockSpec((1, PADDED, D), lambda h: (h, 0, 0)),
            pl.BlockSpec((1, D, N_HASHES), lambda h: (h, 0, 0)),
            pl.BlockSpec((1, PADDED, N_HASHES), lambda h: (h, 0, 0)),
        ],
        out_specs=[
            pl.BlockSpec((1, PADDED, N_HASHES), lambda h: (h, 0, 0)),
            pl.BlockSpec((1, PADDED, N_HASHES), lambda h: (h, 0, 0)),
        ],
        out_shape=[
            jax.ShapeDtypeStruct((N_HEADS, PADDED, N_HASHES), jnp.float32),
            jax.ShapeDtypeStruct((N_HEADS, PADDED, N_HASHES), jnp.float32),
        ],
    )(query, key, alpha, cs_t)
    return out


SBINS = 256
SEG = PADDED // 16


def _sort_pass(keys_src, vals_src, keys_dst, vals_dst, cnt, shift, first):
    lane = lax.iota(jnp.int32, 16)

    zeros = jnp.zeros((16,), jnp.int32)

    def zero_body(t, c):
        plsc.store_scatter(cnt, [t * 16 + lane], zeros)
        return c
    lax.fori_loop(0, SBINS, zero_body, 0)

    def hist_body(t, c):
        idx = lane * SEG + t
        k = plsc.load_gather(keys_src, [idx])
        d = jnp.bitwise_and(lax.shift_right_logical(k, shift), SBINS - 1)
        ci = d * 16 + lane
        cur = plsc.load_gather(cnt, [ci])
        plsc.store_scatter(cnt, [ci], cur + 1)
        return c
    lax.fori_loop(0, SEG, hist_body, 0)

    def scan_body(t, run):
        ci = t * 16 + lane
        v = plsc.load_gather(cnt, [ci])
        s = plsc.cumsum(v)
        plsc.store_scatter(cnt, [ci], s - v + run)
        return run + jnp.sum(v, axis=0)
    lax.fori_loop(0, SBINS, scan_body, jnp.int32(0))

    def perm_body(t, c):
        idx = lane * SEG + t
        k = plsc.load_gather(keys_src, [idx])
        val = idx if first else plsc.load_gather(vals_src, [idx])
        d = jnp.bitwise_and(lax.shift_right_logical(k, shift), SBINS - 1)
        ci = d * 16 + lane
        o = plsc.load_gather(cnt, [ci])
        plsc.store_scatter(cnt, [ci], o + 1)
        plsc.store_scatter(keys_dst, [o], k)
        plsc.store_scatter(vals_dst, [o], val)
        return c
    lax.fori_loop(0, SEG, perm_body, 0)


def _sort_body(keys_hbm, pos_hbm, kf, keys_a, keys_b, vals_a, vals_b, cnt):
    wid = lax.axis_index("s") * 2 + lax.axis_index("c")
    pltpu.sync_copy(keys_hbm.at[wid], kf)
    lane = lax.iota(jnp.int32, 16)

    def xform_body(t, c):
        ci = t * 16 + lane
        x = plsc.load_gather(kf, [ci])
        bits = plsc.bitcast(x, jnp.int32)
        code = jnp.where(bits < 0, jnp.bitwise_not(bits),
                         jnp.bitwise_or(bits, jnp.int32(-2147483648)))
        plsc.store_scatter(keys_a, [ci], code)
        return c
    lax.fori_loop(0, PADDED // 16, xform_body, 0)

    _sort_pass(keys_a, vals_a, keys_b, vals_b, cnt, 0, True)
    _sort_pass(keys_b, vals_b, keys_a, vals_a, cnt, 8, False)
    _sort_pass(keys_a, vals_a, keys_b, vals_b, cnt, 16, False)
    _sort_pass(keys_b, vals_b, keys_a, vals_a, cnt, 24, False)
    pltpu.sync_copy(vals_a, pos_hbm.at[wid])


def _sort_stage(keys):
    run = pl.kernel(
        _sort_body,
        out_type=jax.ShapeDtypeStruct((NW, PADDED), jnp.int32),
        mesh=plsc.VectorSubcoreMesh(core_axis_name="c", subcore_axis_name="s",
                                    num_cores=2, num_subcores=16),
        compiler_params=pltpu.CompilerParams(use_tc_tiling_on_sc=False,
                                             needs_layout_passes=False),
        scratch_types=[
            pltpu.VMEM((PADDED,), jnp.float32),
            pltpu.VMEM((PADDED,), jnp.int32),
            pltpu.VMEM((PADDED,), jnp.int32),
            pltpu.VMEM((PADDED,), jnp.int32),
            pltpu.VMEM((PADDED,), jnp.int32),
            pltpu.VMEM((SBINS * 16,), jnp.int32),
        ],
    )
    return run(keys)


NW = 32
ROWS_PER_W = (N_HASHES * N_HEADS * PADDED) // NW
GCH = 512
DPAD = 48


def _gather_body(q2d, k2d, v2d, qg, kg, sq, sk, sv,
                 idxq_v, idxk_v, rows_q, rows_k, rows_v, sem):
    wid = lax.axis_index("s") * 2 + lax.axis_index("c")
    base = wid * ROWS_PER_W
    for c in range(ROWS_PER_W // GCH):
        off = base + c * GCH
        pltpu.sync_copy(qg.at[pl.ds(off, GCH)], idxq_v)
        pltpu.sync_copy(kg.at[pl.ds(off, GCH)], idxk_v)
        pltpu.async_copy(q2d.at[idxq_v], rows_q, sem).wait()
        pltpu.sync_copy(rows_q, sq.at[pl.ds(off, GCH)])
        pltpu.async_copy(k2d.at[idxk_v], rows_k, sem).wait()
        pltpu.sync_copy(rows_k, sk.at[pl.ds(off, GCH)])
        pltpu.async_copy(v2d.at[idxk_v], rows_v, sem).wait()
        pltpu.sync_copy(rows_v, sv.at[pl.ds(off, GCH)])


def _gather_stage(query, key, value, q_pos, k_pos):
    pad = ((0, 0), (0, 0), (0, DPAD - D))
    q2d = jnp.pad(query, pad).reshape(N_HEADS * PADDED, DPAD)
    k2d = jnp.pad(key, pad).reshape(N_HEADS * PADDED, DPAD)
    v2d = value.reshape(N_HEADS * PADDED, DPH)
    head_off = (jnp.arange(N_HEADS, dtype=jnp.int32) * PADDED)[None, :, None]
    qg = (q_pos + head_off).reshape(-1)
    kg = (k_pos + head_off).reshape(-1)
    run = pl.kernel(
        _gather_body,
        out_type=[
            jax.ShapeDtypeStruct((N_HASHES * N_HEADS * PADDED, DPAD), jnp.float32),
            jax.ShapeDtypeStruct((N_HASHES * N_HEADS * PADDED, DPAD), jnp.float32),
            jax.ShapeDtypeStruct((N_HASHES * N_HEADS * PADDED, DPH), jnp.float32),
        ],
        mesh=plsc.VectorSubcoreMesh(core_axis_name="c", subcore_axis_name="s",
                                    num_cores=2, num_subcores=16),
        compiler_params=pltpu.CompilerParams(use_tc_tiling_on_sc=False),
        scratch_types=[
            pltpu.VMEM((GCH,), jnp.int32),
            pltpu.VMEM((GCH,), jnp.int32),
            pltpu.VMEM((GCH, DPAD), jnp.float32),
            pltpu.VMEM((GCH, DPAD), jnp.float32),
            pltpu.VMEM((GCH, DPH), jnp.float32),
            pltpu.SemaphoreType.DMA,
        ],
    )
    sq2d, sk2d, sv2d = run(q2d, k2d, v2d, qg, kg)
    shp = (N_HASHES, N_HEADS, PADDED)
    return (sq2d.reshape(shp + (DPAD,)), sk2d.reshape(shp + (DPAD,)),
            sv2d.reshape(shp + (DPH,)))


BLOCKS_PER_STEP = 8
ROWS_PER_STEP = BLOCKS_PER_STEP * BLOCK


def _attn_body(q_ref, k_ref, v_ref, o_ref):
    q = q_ref[0, 0].reshape(BLOCKS_PER_STEP, BLOCK, DPAD)
    k = k_ref[0, 0].reshape(BLOCKS_PER_STEP, BLOCK, DPAD)
    v = v_ref[0, 0].reshape(BLOCKS_PER_STEP, BLOCK, DPH)
    qn = jnp.sum(q * q, axis=-1)
    kn = jnp.sum(k * k, axis=-1)
    qk = jax.lax.dot_general(q, k, (((2,), (2,)), ((0,), (0,))),
                             preferred_element_type=jnp.float32, precision=jax.lax.Precision.HIGHEST)
    d2 = qn[:, :, None] + kn[:, None, :] - 2.0 * qk
    w = jnp.exp(-0.5 * d2)
    o = jax.lax.dot_general(w, v, (((2,), (1,)), ((0,), (0,))),
                            preferred_element_type=jnp.float32, precision=jax.lax.Precision.HIGHEST)
    o_ref[0, 0] = o


def _attn_stage(sq, sk, sv):
    n_steps = PADDED // ROWS_PER_STEP
    out = pl.pallas_call(
        _attn_body,
        grid=(N_HASHES, N_HEADS, n_steps),
        in_specs=[
            pl.BlockSpec((1, 1, ROWS_PER_STEP, DPAD), lambda a, h, b: (a, h, b, 0)),
            pl.BlockSpec((1, 1, ROWS_PER_STEP, DPAD), lambda a, h, b: (a, h, b, 0)),
            pl.BlockSpec((1, 1, ROWS_PER_STEP, DPH), lambda a, h, b: (a, h, b, 0)),
        ],
        out_specs=pl.BlockSpec((1, 1, BLOCKS_PER_STEP, BLOCK, DPH),
                               lambda a, h, b: (a, h, b, 0, 0)),
        out_shape=jax.ShapeDtypeStruct(
            (N_HASHES, N_HEADS, PADDED // BLOCK, BLOCK, DPH), jnp.float32),
    )(sq, sk, sv)
    return out


def kernel(query, key, value, combined_shifts, alpha):
    q_hashed = jnp.einsum('hpd,hdn->hpn', query, alpha).transpose(2, 0, 1)
    k_hashed = jnp.einsum('hpd,hdn->hpn', key, alpha).transpose(2, 0, 1)
    max_hash_shift = jnp.maximum(q_hashed.max(-1, keepdims=True),
                                 k_hashed.max(-1, keepdims=True))
    min_hash_shift = jnp.minimum(q_hashed.min(-1, keepdims=True),
                                 k_hashed.min(-1, keepdims=True))
    hash_shift = max_hash_shift - min_hash_shift
    cs = combined_shifts.astype(jnp.float32) * hash_shift
    q_keys = q_hashed + cs
    k_keys = k_hashed + cs
    all_keys = jnp.concatenate(
        [q_keys.reshape(-1, PADDED), k_keys.reshape(-1, PADDED)], axis=0)
    all_pos = _sort_stage(all_keys)
    q_pos = all_pos[:N_HASHES * N_HEADS].reshape(N_HASHES, N_HEADS, PADDED)
    k_pos = all_pos[N_HASHES * N_HEADS:].reshape(N_HASHES, N_HEADS, PADDED)
    sq, sk, sv = _gather_stage(query, key, value, q_pos, k_pos)
    return _attn_stage(sq, sk, sv)


# speedup vs baseline: 1.5695x; 1.1134x over previous
import functools

import jax
import jax.numpy as jnp
from jax import lax
from jax.experimental import pallas as pl
from jax.experimental.pallas import tpu as pltpu
from jax.experimental.pallas import tpu_sc as plsc

N_HASHES = 2
N_HEADS = 8
PADDED = 4096
BLOCK = 64
DPH = 32
CDIM = 3
D = DPH + CDIM


def _hash_body(q_ref, k_ref, al_ref, cs_ref, qh_ref, kh_ref):
    q = q_ref[0]
    k = k_ref[0]
    al = al_ref[0]
    cs = cs_ref[0].astype(jnp.float32)
    qh = jax.lax.dot_general(q, al, (((1,), (0,)), ((), ())),
                             preferred_element_type=jnp.float32, precision=jax.lax.Precision.HIGHEST)
    kh = jax.lax.dot_general(k, al, (((1,), (0,)), ((), ())),
                             preferred_element_type=jnp.float32, precision=jax.lax.Precision.HIGHEST)
    mx = jnp.maximum(jnp.max(qh, axis=0), jnp.max(kh, axis=0))
    mn = jnp.minimum(jnp.min(qh, axis=0), jnp.min(kh, axis=0))
    shift = cs * (mx - mn)[None, :]
    qh_ref[0] = qh + shift
    kh_ref[0] = kh + shift


def _hash_stage(query, key, alpha, cs_t):
    out = pl.pallas_call(
        _hash_body,
        grid=(N_HEADS,),
        in_specs=[
            pl.BlockSpec((1, PADDED, D), lambda h: (h, 0, 0)),
            pl.BlockSpec((1, PADDED, D), lambda h: (h, 0, 0)),
            pl.BlockSpec((1, D, N_HASHES), lambda h: (h, 0, 0)),
            pl.BlockSpec((1, PADDED, N_HASHES), lambda h: (h, 0, 0)),
        ],
        out_specs=[
            pl.BlockSpec((1, PADDED, N_HASHES), lambda h: (h, 0, 0)),
            pl.BlockSpec((1, PADDED, N_HASHES), lambda h: (h, 0, 0)),
        ],
        out_shape=[
            jax.ShapeDtypeStruct((N_HEADS, PADDED, N_HASHES), jnp.float32),
            jax.ShapeDtypeStruct((N_HEADS, PADDED, N_HASHES), jnp.float32),
        ],
    )(query, key, alpha, cs_t)
    return out


SBINS = 256
SEG = PADDED // 16


def _sort_pass(keys_src, vals_src, keys_dst, vals_dst, cnt, shift, first):
    lane = lax.iota(jnp.int32, 16)

    zeros = jnp.zeros((16,), jnp.int32)

    def zero_body(t, c):
        plsc.store_scatter(cnt, [t * 16 + lane], zeros)
        return c
    lax.fori_loop(0, SBINS, zero_body, 0)

    def hist_body(t, c):
        idx = lane * SEG + t
        k = plsc.load_gather(keys_src, [idx])
        d = jnp.bitwise_and(lax.shift_right_logical(k, shift), SBINS - 1)
        ci = d * 16 + lane
        cur = plsc.load_gather(cnt, [ci])
        plsc.store_scatter(cnt, [ci], cur + 1)
        return c
    lax.fori_loop(0, SEG, hist_body, 0)

    def scan_body(t, run):
        ci = t * 16 + lane
        v = plsc.load_gather(cnt, [ci])
        s = plsc.cumsum(v)
        plsc.store_scatter(cnt, [ci], s - v + run)
        return run + jnp.sum(v, axis=0)
    lax.fori_loop(0, SBINS, scan_body, jnp.int32(0))

    def perm_body(t, c):
        idx = lane * SEG + t
        k = plsc.load_gather(keys_src, [idx])
        val = idx if first else plsc.load_gather(vals_src, [idx])
        d = jnp.bitwise_and(lax.shift_right_logical(k, shift), SBINS - 1)
        ci = d * 16 + lane
        o = plsc.load_gather(cnt, [ci])
        plsc.store_scatter(cnt, [ci], o + 1)
        plsc.store_scatter(keys_dst, [o], k)
        plsc.store_scatter(vals_dst, [o], val)
        return c
    lax.fori_loop(0, SEG, perm_body, 0)


def _sort_body(keys_hbm, pos_hbm, kf, keys_a, keys_b, vals_a, vals_b, cnt):
    wid = lax.axis_index("s") * 2 + lax.axis_index("c")
    pltpu.sync_copy(keys_hbm.at[wid], kf)
    lane = lax.iota(jnp.int32, 16)

    def xform_body(t, c):
        ci = t * 16 + lane
        x = plsc.load_gather(kf, [ci])
        bits = plsc.bitcast(x, jnp.int32)
        code = jnp.where(bits < 0, jnp.bitwise_not(bits),
                         jnp.bitwise_or(bits, jnp.int32(-2147483648)))
        plsc.store_scatter(keys_a, [ci], code)
        return c
    lax.fori_loop(0, PADDED // 16, xform_body, 0)

    _sort_pass(keys_a, vals_a, keys_b, vals_b, cnt, 0, True)
    _sort_pass(keys_b, vals_b, keys_a, vals_a, cnt, 8, False)
    _sort_pass(keys_a, vals_a, keys_b, vals_b, cnt, 16, False)
    _sort_pass(keys_b, vals_b, keys_a, vals_a, cnt, 24, False)
    pltpu.sync_copy(vals_a, pos_hbm.at[wid])


def _sort_stage(keys):
    run = pl.kernel(
        _sort_body,
        out_type=jax.ShapeDtypeStruct((NW, PADDED), jnp.int32),
        mesh=plsc.VectorSubcoreMesh(core_axis_name="c", subcore_axis_name="s",
                                    num_cores=2, num_subcores=16),
        compiler_params=pltpu.CompilerParams(use_tc_tiling_on_sc=False,
                                             needs_layout_passes=False),
        scratch_types=[
            pltpu.VMEM((PADDED,), jnp.float32),
            pltpu.VMEM((PADDED,), jnp.int32),
            pltpu.VMEM((PADDED,), jnp.int32),
            pltpu.VMEM((PADDED,), jnp.int32),
            pltpu.VMEM((PADDED,), jnp.int32),
            pltpu.VMEM((SBINS * 16,), jnp.int32),
        ],
    )
    return run(keys)


NW = 32
ROWS_PER_W = (N_HASHES * N_HEADS * PADDED) // NW
GCH = 1024
DPAD = 48


def _gather_body(q2d, k2d, v2d, qg, kg, sq, sk, sv,
                 idx_v, rows_qk, rows_v, sem):
    wid = lax.axis_index("s") * 2 + lax.axis_index("c")
    base = wid * ROWS_PER_W
    for c in range(ROWS_PER_W // GCH):
        off = base + c * GCH
        pltpu.sync_copy(qg.at[pl.ds(off, GCH)], idx_v)
        pltpu.async_copy(q2d.at[idx_v], rows_qk, sem).wait()
        pltpu.sync_copy(rows_qk, sq.at[pl.ds(off, GCH)])
    for c in range(ROWS_PER_W // GCH):
        off = base + c * GCH
        pltpu.sync_copy(kg.at[pl.ds(off, GCH)], idx_v)
        pltpu.async_copy(k2d.at[idx_v], rows_qk, sem).wait()
        pltpu.sync_copy(rows_qk, sk.at[pl.ds(off, GCH)])
        pltpu.async_copy(v2d.at[idx_v], rows_v, sem).wait()
        pltpu.sync_copy(rows_v, sv.at[pl.ds(off, GCH)])


def _gather_stage(query, key, value, q_pos, k_pos):
    pad = ((0, 0), (0, 0), (0, DPAD - D))
    q2d = jnp.pad(query, pad).reshape(N_HEADS * PADDED, DPAD)
    k2d = jnp.pad(key, pad).reshape(N_HEADS * PADDED, DPAD)
    v2d = value.reshape(N_HEADS * PADDED, DPH)
    head_off = (jnp.arange(N_HEADS, dtype=jnp.int32) * PADDED)[None, :, None]
    qg = (q_pos + head_off).reshape(-1)
    kg = (k_pos + head_off).reshape(-1)
    run = pl.kernel(
        _gather_body,
        out_type=[
            jax.ShapeDtypeStruct((N_HASHES * N_HEADS * PADDED, DPAD), jnp.float32),
            jax.ShapeDtypeStruct((N_HASHES * N_HEADS * PADDED, DPAD), jnp.float32),
            jax.ShapeDtypeStruct((N_HASHES * N_HEADS * PADDED, DPH), jnp.float32),
        ],
        mesh=plsc.VectorSubcoreMesh(core_axis_name="c", subcore_axis_name="s",
                                    num_cores=2, num_subcores=16),
        compiler_params=pltpu.CompilerParams(use_tc_tiling_on_sc=False),
        scratch_types=[
            pltpu.VMEM((GCH,), jnp.int32),
            pltpu.VMEM((GCH, DPAD), jnp.float32),
            pltpu.VMEM((GCH, DPH), jnp.float32),
            pltpu.SemaphoreType.DMA,
        ],
    )
    sq2d, sk2d, sv2d = run(q2d, k2d, v2d, qg, kg)
    shp = (N_HASHES, N_HEADS, PADDED)
    return (sq2d.reshape(shp + (DPAD,)), sk2d.reshape(shp + (DPAD,)),
            sv2d.reshape(shp + (DPH,)))


BLOCKS_PER_STEP = 16
ROWS_PER_STEP = BLOCKS_PER_STEP * BLOCK


def _attn_body(q_ref, k_ref, v_ref, o_ref):
    q = q_ref[0, 0].reshape(BLOCKS_PER_STEP, BLOCK, DPAD)
    k = k_ref[0, 0].reshape(BLOCKS_PER_STEP, BLOCK, DPAD)
    v = v_ref[0, 0].reshape(BLOCKS_PER_STEP, BLOCK, DPH)
    qn = jnp.sum(q * q, axis=-1)
    kn = jnp.sum(k * k, axis=-1)
    qk = jax.lax.dot_general(q, k, (((2,), (2,)), ((0,), (0,))),
                             preferred_element_type=jnp.float32, precision=jax.lax.Precision.HIGHEST)
    d2 = qn[:, :, None] + kn[:, None, :] - 2.0 * qk
    w = jnp.exp(-0.5 * d2)
    o = jax.lax.dot_general(w, v, (((2,), (1,)), ((0,), (0,))),
                            preferred_element_type=jnp.float32, precision=jax.lax.Precision.HIGHEST)
    o_ref[0, 0] = o


def _attn_stage(sq, sk, sv):
    n_steps = PADDED // ROWS_PER_STEP
    out = pl.pallas_call(
        _attn_body,
        grid=(N_HASHES, N_HEADS, n_steps),
        in_specs=[
            pl.BlockSpec((1, 1, ROWS_PER_STEP, DPAD), lambda a, h, b: (a, h, b, 0)),
            pl.BlockSpec((1, 1, ROWS_PER_STEP, DPAD), lambda a, h, b: (a, h, b, 0)),
            pl.BlockSpec((1, 1, ROWS_PER_STEP, DPH), lambda a, h, b: (a, h, b, 0)),
        ],
        out_specs=pl.BlockSpec((1, 1, BLOCKS_PER_STEP, BLOCK, DPH),
                               lambda a, h, b: (a, h, b, 0, 0)),
        out_shape=jax.ShapeDtypeStruct(
            (N_HASHES, N_HEADS, PADDED // BLOCK, BLOCK, DPH), jnp.float32),
    )(sq, sk, sv)
    return out


def kernel(query, key, value, combined_shifts, alpha):
    q_hashed = jnp.einsum('hpd,hdn->hpn', query, alpha).transpose(2, 0, 1)
    k_hashed = jnp.einsum('hpd,hdn->hpn', key, alpha).transpose(2, 0, 1)
    max_hash_shift = jnp.maximum(q_hashed.max(-1, keepdims=True),
                                 k_hashed.max(-1, keepdims=True))
    min_hash_shift = jnp.minimum(q_hashed.min(-1, keepdims=True),
                                 k_hashed.min(-1, keepdims=True))
    hash_shift = max_hash_shift - min_hash_shift
    cs = combined_shifts.astype(jnp.float32) * hash_shift
    q_keys = q_hashed + cs
    k_keys = k_hashed + cs
    all_keys = jnp.concatenate(
        [q_keys.reshape(-1, PADDED), k_keys.reshape(-1, PADDED)], axis=0)
    all_pos = _sort_stage(all_keys)
    q_pos = all_pos[:N_HASHES * N_HEADS].reshape(N_HASHES, N_HEADS, PADDED)
    k_pos = all_pos[N_HASHES * N_HEADS:].reshape(N_HASHES, N_HEADS, PADDED)
    sq, sk, sv = _gather_stage(query, key, value, q_pos, k_pos)
    return _attn_stage(sq, sk, sv)


# speedup vs baseline: 1.5705x; 1.0006x over previous
import jax
import jax.numpy as jnp
from jax import lax
from jax.experimental import pallas as pl
from jax.experimental.pallas import tpu as pltpu
from jax.experimental.pallas import tpu_sc as plsc

N_HASHES = 2
N_HEADS = 8
PADDED = 4096
BLOCK = 64
DPH = 32
CDIM = 3
D = DPH + CDIM


SBINS = 256
SEG = PADDED // 16


def _sort_pass(keys_src, vals_src, keys_dst, vals_dst, cnt, shift, first):
    lane = lax.iota(jnp.int32, 16)

    zeros = jnp.zeros((16,), jnp.int32)

    def zero_body(t, c):
        plsc.store_scatter(cnt, [t * 16 + lane], zeros)
        return c
    lax.fori_loop(0, SBINS, zero_body, 0)

    def hist_body(t, c):
        idx = lane * SEG + t
        k = plsc.load_gather(keys_src, [idx])
        d = jnp.bitwise_and(lax.shift_right_logical(k, shift), SBINS - 1)
        ci = d * 16 + lane
        cur = plsc.load_gather(cnt, [ci])
        plsc.store_scatter(cnt, [ci], cur + 1)
        return c
    lax.fori_loop(0, SEG, hist_body, 0)

    def scan_body(t, run):
        ci = t * 16 + lane
        v = plsc.load_gather(cnt, [ci])
        s = plsc.cumsum(v)
        plsc.store_scatter(cnt, [ci], s - v + run)
        return run + jnp.sum(v, axis=0)
    lax.fori_loop(0, SBINS, scan_body, jnp.int32(0))

    def perm_body(t, c):
        idx = lane * SEG + t
        k = plsc.load_gather(keys_src, [idx])
        val = idx if first else plsc.load_gather(vals_src, [idx])
        d = jnp.bitwise_and(lax.shift_right_logical(k, shift), SBINS - 1)
        ci = d * 16 + lane
        o = plsc.load_gather(cnt, [ci])
        plsc.store_scatter(cnt, [ci], o + 1)
        plsc.store_scatter(keys_dst, [o], k)
        plsc.store_scatter(vals_dst, [o], val)
        return c
    lax.fori_loop(0, SEG, perm_body, 0)


def _sort_body(keys_hbm, pos_hbm, kf, keys_a, keys_b, vals_a, vals_b, cnt):
    wid = lax.axis_index("s") * 2 + lax.axis_index("c")
    pltpu.sync_copy(keys_hbm.at[wid], kf)
    lane = lax.iota(jnp.int32, 16)

    def xform_body(t, c):
        ci = t * 16 + lane
        x = plsc.load_gather(kf, [ci])
        bits = plsc.bitcast(x, jnp.int32)
        code = jnp.where(bits < 0, jnp.bitwise_not(bits),
                         jnp.bitwise_or(bits, jnp.int32(-2147483648)))
        plsc.store_scatter(keys_a, [ci], code)
        return c
    lax.fori_loop(0, PADDED // 16, xform_body, 0)

    _sort_pass(keys_a, vals_a, keys_b, vals_b, cnt, 0, True)
    _sort_pass(keys_b, vals_b, keys_a, vals_a, cnt, 8, False)
    _sort_pass(keys_a, vals_a, keys_b, vals_b, cnt, 16, False)
    _sort_pass(keys_b, vals_b, keys_a, vals_a, cnt, 24, False)
    pltpu.sync_copy(vals_a, pos_hbm.at[wid])


def _sort_stage(keys):
    run = pl.kernel(
        _sort_body,
        out_type=jax.ShapeDtypeStruct((NW, PADDED), jnp.int32),
        mesh=plsc.VectorSubcoreMesh(core_axis_name="c", subcore_axis_name="s",
                                    num_cores=2, num_subcores=16),
        compiler_params=pltpu.CompilerParams(use_tc_tiling_on_sc=False,
                                             needs_layout_passes=False),
        scratch_types=[
            pltpu.VMEM((PADDED,), jnp.float32),
            pltpu.VMEM((PADDED,), jnp.int32),
            pltpu.VMEM((PADDED,), jnp.int32),
            pltpu.VMEM((PADDED,), jnp.int32),
            pltpu.VMEM((PADDED,), jnp.int32),
            pltpu.VMEM((SBINS * 16,), jnp.int32),
        ],
    )
    return run(keys)


NW = 32
ROWS_PER_W = (N_HASHES * N_HEADS * PADDED) // NW
GCH = 1024
DPAD = 48


def _gather_body(q2d, k2d, v2d, qg, kg, sq, sk, sv,
                 idx_v, rows_qk, rows_v, sem):
    wid = lax.axis_index("s") * 2 + lax.axis_index("c")
    base = wid * ROWS_PER_W
    for c in range(ROWS_PER_W // GCH):
        off = base + c * GCH
        pltpu.sync_copy(qg.at[pl.ds(off, GCH)], idx_v)
        pltpu.async_copy(q2d.at[idx_v], rows_qk, sem).wait()
        pltpu.sync_copy(rows_qk, sq.at[pl.ds(off, GCH)])
    for c in range(ROWS_PER_W // GCH):
        off = base + c * GCH
        pltpu.sync_copy(kg.at[pl.ds(off, GCH)], idx_v)
        pltpu.async_copy(k2d.at[idx_v], rows_qk, sem).wait()
        pltpu.sync_copy(rows_qk, sk.at[pl.ds(off, GCH)])
        pltpu.async_copy(v2d.at[idx_v], rows_v, sem).wait()
        pltpu.sync_copy(rows_v, sv.at[pl.ds(off, GCH)])


def _gather_stage(query, key, value, q_pos, k_pos):
    pad = ((0, 0), (0, 0), (0, DPAD - D))
    q2d = jnp.pad(query, pad).reshape(N_HEADS * PADDED, DPAD)
    k2d = jnp.pad(key, pad).reshape(N_HEADS * PADDED, DPAD)
    v2d = value.reshape(N_HEADS * PADDED, DPH)
    head_off = (jnp.arange(N_HEADS, dtype=jnp.int32) * PADDED)[None, :, None]
    qg = (q_pos + head_off).reshape(-1)
    kg = (k_pos + head_off).reshape(-1)
    run = pl.kernel(
        _gather_body,
        out_type=[
            jax.ShapeDtypeStruct((N_HASHES * N_HEADS * PADDED, DPAD), jnp.float32),
            jax.ShapeDtypeStruct((N_HASHES * N_HEADS * PADDED, DPAD), jnp.float32),
            jax.ShapeDtypeStruct((N_HASHES * N_HEADS * PADDED, DPH), jnp.float32),
        ],
        mesh=plsc.VectorSubcoreMesh(core_axis_name="c", subcore_axis_name="s",
                                    num_cores=2, num_subcores=16),
        compiler_params=pltpu.CompilerParams(use_tc_tiling_on_sc=False),
        scratch_types=[
            pltpu.VMEM((GCH,), jnp.int32),
            pltpu.VMEM((GCH, DPAD), jnp.float32),
            pltpu.VMEM((GCH, DPH), jnp.float32),
            pltpu.SemaphoreType.DMA,
        ],
    )
    sq2d, sk2d, sv2d = run(q2d, k2d, v2d, qg, kg)
    shp = (N_HASHES, N_HEADS, PADDED)
    return (sq2d.reshape(shp + (DPAD,)), sk2d.reshape(shp + (DPAD,)),
            sv2d.reshape(shp + (DPH,)))


BLOCKS_PER_STEP = 16
ROWS_PER_STEP = BLOCKS_PER_STEP * BLOCK


def _attn_body(q_ref, k_ref, v_ref, o_ref):
    q = q_ref[0, 0].reshape(BLOCKS_PER_STEP, BLOCK, DPAD)
    k = k_ref[0, 0].reshape(BLOCKS_PER_STEP, BLOCK, DPAD)
    v = v_ref[0, 0].reshape(BLOCKS_PER_STEP, BLOCK, DPH)
    qn = jnp.sum(q * q, axis=-1)
    kn = jnp.sum(k * k, axis=-1)
    qk = jax.lax.dot_general(q, k, (((2,), (2,)), ((0,), (0,))),
                             preferred_element_type=jnp.float32, precision=jax.lax.Precision.HIGHEST)
    d2 = qn[:, :, None] + kn[:, None, :] - 2.0 * qk
    w = jnp.exp(-0.5 * d2)
    o = jax.lax.dot_general(w, v, (((2,), (1,)), ((0,), (0,))),
                            preferred_element_type=jnp.float32, precision=jax.lax.Precision.HIGHEST)
    o_ref[0, 0] = o


def _attn_stage(sq, sk, sv):
    n_steps = PADDED // ROWS_PER_STEP
    out = pl.pallas_call(
        _attn_body,
        grid=(N_HASHES, N_HEADS, n_steps),
        in_specs=[
            pl.BlockSpec((1, 1, ROWS_PER_STEP, DPAD), lambda a, h, b: (a, h, b, 0)),
            pl.BlockSpec((1, 1, ROWS_PER_STEP, DPAD), lambda a, h, b: (a, h, b, 0)),
            pl.BlockSpec((1, 1, ROWS_PER_STEP, DPH), lambda a, h, b: (a, h, b, 0)),
        ],
        out_specs=pl.BlockSpec((1, 1, BLOCKS_PER_STEP, BLOCK, DPH),
                               lambda a, h, b: (a, h, b, 0, 0)),
        out_shape=jax.ShapeDtypeStruct(
            (N_HASHES, N_HEADS, PADDED // BLOCK, BLOCK, DPH), jnp.float32),
    )(sq, sk, sv)
    return out


def kernel(query, key, value, combined_shifts, alpha):
    q_hashed = jnp.einsum('hpd,hdn->hpn', query, alpha).transpose(2, 0, 1)
    k_hashed = jnp.einsum('hpd,hdn->hpn', key, alpha).transpose(2, 0, 1)
    max_hash_shift = jnp.maximum(q_hashed.max(-1, keepdims=True),
                                 k_hashed.max(-1, keepdims=True))
    min_hash_shift = jnp.minimum(q_hashed.min(-1, keepdims=True),
                                 k_hashed.min(-1, keepdims=True))
    hash_shift = max_hash_shift - min_hash_shift
    cs = combined_shifts.astype(jnp.float32) * hash_shift
    q_keys = q_hashed + cs
    k_keys = k_hashed + cs
    all_keys = jnp.concatenate(
        [q_keys.reshape(-1, PADDED), k_keys.reshape(-1, PADDED)], axis=0)
    all_pos = _sort_stage(all_keys)
    q_pos = all_pos[:N_HASHES * N_HEADS].reshape(N_HASHES, N_HEADS, PADDED)
    k_pos = all_pos[N_HASHES * N_HEADS:].reshape(N_HASHES, N_HEADS, PADDED)
    sq, sk, sv = _gather_stage(query, key, value, q_pos, k_pos)
    return _attn_stage(sq, sk, sv)
